# baseline (speedup 1.0000x reference)
import jax
import jax.numpy as jnp
from jax import lax
from jax.experimental import pallas as pl
from jax.experimental.pallas import tpu as pltpu
from jax.experimental.pallas import tpu_sc as plsc

_VOCAB = 1000
_DIM = 64
_CTX = 20
_NEG = 5
_NW = 32
_NP = _DIM // 2
_PH = _NP // 2
_PV = 1024
_LANES = 16


def _pack_pairs(Wt):
    b = Wt.astype(jnp.bfloat16).reshape(_NP, 2, -1)
    u = lax.bitcast_convert_type(b, jnp.uint16).astype(jnp.uint32)
    w = u[:, 0, :] | (u[:, 1, :] << jnp.uint32(16))
    return lax.bitcast_convert_type(w, jnp.int32).reshape(-1)


def _fold_pairs(acc_bf32):
    w = plsc.bitcast(acc_bf32, jnp.int32)
    lo = plsc.bitcast(w << 16, jnp.float32)
    hi = plsc.bitcast(w & jnp.int32(-65536), jnp.float32)
    return lo + hi


def _sc_body(ctxw_hbm, embw_hbm, ctxidx_hbm, tgt_hbm, r_hbm, pcum_hbm, out_hbm,
             ctxw_v, embw_v, p_v, idx_v, tgt_v, r_v, sc_v):
    wid = lax.axis_index("s") * 2 + lax.axis_index("c")
    rg = tgt_v.shape[0]
    base = wid * rg
    pltpu.sync_copy(ctxw_hbm, ctxw_v)
    pltpu.sync_copy(embw_hbm, embw_v)
    pltpu.sync_copy(pcum_hbm, p_v)
    pltpu.sync_copy(ctxidx_hbm.at[:, pl.ds(base, rg)], idx_v)
    pltpu.sync_copy(tgt_hbm.at[pl.ds(base, rg)], tgt_v)
    pltpu.sync_copy(r_hbm.at[:, pl.ds(base, rg)], r_v)

    zero_bf = jnp.zeros((2 * _LANES,), jnp.bfloat16)

    @plsc.parallel_loop(0, rg // _LANES)
    def blk(i):
        b0 = i * _LANES
        ti = tgt_v[pl.ds(b0, _LANES)]
        rs = [r_v[k, pl.ds(b0, _LANES)] for k in range(_NEG)]
        nis = [jnp.zeros((_LANES,), jnp.int32) for _ in range(_NEG)]
        posp = zero_bf
        negp = [zero_bf for _ in range(_NEG)]
        s = _PV // 2
        for half in range(2):
            cvp = [zero_bf for _ in range(_PH)]
            iota16 = lax.iota(jnp.int32, _LANES)
            for c in range(_CTX):
                ci = idx_v[c, pl.ds(b0, _LANES)]
                ci = (ci & jnp.int32(-16)) + iota16
                for p in range(_PH):
                    w = plsc.load_gather(
                        ctxw_v, [ci + ((half * _PH + p) * _VOCAB)])
                    cvp[p] = cvp[p] + plsc.bitcast(w, jnp.bfloat16)
                if half == 0 and c % 2 == 1:
                    for k in range(_NEG):
                        val = plsc.load_gather(p_v, [nis[k] + (s - 1)])
                        nis[k] = nis[k] + jnp.where(
                            val < rs[k], s, 0).astype(jnp.int32)
                    s //= 2
            for p in range(_PH):
                off = (half * _PH + p) * _VOCAB
                tw = plsc.load_gather(embw_v, [ti + off])
                posp = posp + cvp[p] * plsc.bitcast(tw, jnp.bfloat16)
                for k in range(_NEG):
                    nw = plsc.load_gather(embw_v, [nis[k] + off])
                    negp[k] = negp[k] + cvp[p] * plsc.bitcast(nw, jnp.bfloat16)
        inv_ctx = jnp.float32(1.0) / jnp.float32(_CTX)
        sc_v[0, pl.ds(b0, _LANES)] = _fold_pairs(posp) * inv_ctx
        for k in range(_NEG):
            sc_v[1 + k, pl.ds(b0, _LANES)] = _fold_pairs(negp[k]) * inv_ctx

    pltpu.sync_copy(sc_v, out_hbm.at[:, pl.ds(base, rg)])


def _tc_body(s_ref, o_ref):
    x = s_ref[...]
    pos = x[0:1, :]
    neg = x[1:6, :]
    tot = jax.nn.log_sigmoid(pos) + jnp.sum(
        jax.nn.log_sigmoid(-neg), axis=0, keepdims=True)
    o_ref[:, :] = jnp.reshape(-jnp.mean(tot), (1, 1))


def kernel(context, target, emb_W, ctx_W, word_freq):
    B = context.shape[0]
    rg = B // _NW
    context = context.astype(jnp.int32)
    target = target.astype(jnp.int32)
    probs = jnp.power(word_freq, 0.75)
    probs = probs / probs.sum()
    p_cuml = jnp.cumsum(probs)
    u = jax.random.uniform(jax.random.key(1), (B, _NEG), dtype=p_cuml.dtype)
    r = p_cuml[-1] * (1.0 - u)
    p_pad = jnp.concatenate(
        [p_cuml, jnp.full((_PV - _VOCAB,), 2.0, jnp.float32)])

    mesh = plsc.VectorSubcoreMesh(core_axis_name="c", subcore_axis_name="s")
    sc = pl.kernel(
        _sc_body,
        out_type=jax.ShapeDtypeStruct((6, B), jnp.float32),
        mesh=mesh,
        compiler_params=pltpu.CompilerParams(needs_layout_passes=False),
        scratch_types=[
            pltpu.VMEM((_NP * _VOCAB,), jnp.int32),
            pltpu.VMEM((_NP * _VOCAB,), jnp.int32),
            pltpu.VMEM((_PV,), jnp.float32),
            pltpu.VMEM((_CTX, rg), jnp.int32),
            pltpu.VMEM((rg,), jnp.int32),
            pltpu.VMEM((_NEG, rg), jnp.float32),
            pltpu.VMEM((6, rg), jnp.float32),
        ],
    )
    scores = sc(_pack_pairs(ctx_W.T), _pack_pairs(emb_W.T),
                context.T, target, r.T, p_pad)

    loss = pl.pallas_call(
        _tc_body,
        out_shape=jax.ShapeDtypeStruct((1, 1), jnp.float32),
    )(scores)
    return loss[0, 0]


# speedup vs baseline: 1.2379x; 1.2379x over previous
import jax
import jax.numpy as jnp
from jax import lax
from jax.experimental import pallas as pl
from jax.experimental.pallas import tpu as pltpu
from jax.experimental.pallas import tpu_sc as plsc

_VOCAB = 1000
_DIM = 64
_CTX = 20
_NEG = 5
_NW = 32
_NP = _DIM // 2
_PH = _NP // 2
_PV = 1024
_LANES = 16


def _pack_pairs(Wt):
    b = Wt.astype(jnp.bfloat16).reshape(_NP, 2, -1)
    u = lax.bitcast_convert_type(b, jnp.uint16).astype(jnp.uint32)
    w = u[:, 0, :] | (u[:, 1, :] << jnp.uint32(16))
    return lax.bitcast_convert_type(w, jnp.int32).reshape(-1)


def _fold_pairs(acc_bf32):
    w = plsc.bitcast(acc_bf32, jnp.int32)
    lo = plsc.bitcast(w << 16, jnp.float32)
    hi = plsc.bitcast(w & jnp.int32(-65536), jnp.float32)
    return lo + hi


def _sc_body(ctxw_hbm, embw_hbm, ctxidx_hbm, tgt_hbm, r_hbm, pcum_hbm, out_hbm,
             ctxw_v, embw_v, p_v, idx_v, tgt_v, r_v, sc_v):
    wid = lax.axis_index("s") * 2 + lax.axis_index("c")
    rg = tgt_v.shape[0]
    base = wid * rg
    pltpu.sync_copy(ctxw_hbm, ctxw_v)
    pltpu.sync_copy(embw_hbm, embw_v)
    pltpu.sync_copy(pcum_hbm, p_v)
    pltpu.sync_copy(ctxidx_hbm.at[:, pl.ds(base, rg)], idx_v)
    pltpu.sync_copy(tgt_hbm.at[pl.ds(base, rg)], tgt_v)
    pltpu.sync_copy(r_hbm.at[:, pl.ds(base, rg)], r_v)

    zero_bf = jnp.zeros((2 * _LANES,), jnp.bfloat16)
    kstep = jnp.full((_LANES,), _VOCAB, jnp.int32)

    @plsc.parallel_loop(0, rg // _LANES)
    def blk(i):
        b0 = i * _LANES
        ti = tgt_v[pl.ds(b0, _LANES)]
        rs = [r_v[k, pl.ds(b0, _LANES)] for k in range(_NEG)]
        nis = [jnp.zeros((_LANES,), jnp.int32) for _ in range(_NEG)]
        posp = zero_bf
        negp = [zero_bf for _ in range(_NEG)]
        s = _PV // 2
        for half in range(2):
            cvp = [zero_bf for _ in range(_PH)]
            for c in range(_CTX):
                ci = idx_v[c, pl.ds(b0, _LANES)]
                addr = ci + (half * _PH * _VOCAB) if half else ci
                for p in range(_PH):
                    w = plsc.load_gather(ctxw_v, [addr])
                    if p + 1 < _PH:
                        addr = addr + kstep
                    cvp[p] = cvp[p] + plsc.bitcast(w, jnp.bfloat16)
                if half == 0 and c % 2 == 1:
                    for k in range(_NEG):
                        val = plsc.load_gather(p_v, [nis[k] + (s - 1)])
                        nis[k] = nis[k] + jnp.where(
                            val < rs[k], s, 0).astype(jnp.int32)
                    s //= 2
            toff = ti + (half * _PH * _VOCAB) if half else ti
            noff = [ni + (half * _PH * _VOCAB) if half else ni for ni in nis]
            for p in range(_PH):
                tw = plsc.load_gather(embw_v, [toff])
                posp = posp + cvp[p] * plsc.bitcast(tw, jnp.bfloat16)
                for k in range(_NEG):
                    nw = plsc.load_gather(embw_v, [noff[k]])
                    negp[k] = negp[k] + cvp[p] * plsc.bitcast(nw, jnp.bfloat16)
                if p + 1 < _PH:
                    toff = toff + kstep
                    noff = [x + kstep for x in noff]
        inv_ctx = jnp.float32(1.0) / jnp.float32(_CTX)
        sc_v[0, pl.ds(b0, _LANES)] = _fold_pairs(posp) * inv_ctx
        for k in range(_NEG):
            sc_v[1 + k, pl.ds(b0, _LANES)] = _fold_pairs(negp[k]) * inv_ctx

    pltpu.sync_copy(sc_v, out_hbm.at[:, pl.ds(base, rg)])


def _tc_body(s_ref, o_ref):
    x = s_ref[...]
    pos = x[0:1, :]
    neg = x[1:6, :]
    tot = jax.nn.log_sigmoid(pos) + jnp.sum(
        jax.nn.log_sigmoid(-neg), axis=0, keepdims=True)
    o_ref[:, :] = jnp.reshape(-jnp.mean(tot), (1, 1))


def kernel(context, target, emb_W, ctx_W, word_freq):
    B = context.shape[0]
    rg = B // _NW
    context = context.astype(jnp.int32)
    target = target.astype(jnp.int32)
    probs = jnp.power(word_freq, 0.75)
    probs = probs / probs.sum()
    p_cuml = jnp.cumsum(probs)
    u = jax.random.uniform(jax.random.key(1), (B, _NEG), dtype=p_cuml.dtype)
    r = p_cuml[-1] * (1.0 - u)
    p_pad = jnp.concatenate(
        [p_cuml, jnp.full((_PV - _VOCAB,), 2.0, jnp.float32)])

    mesh = plsc.VectorSubcoreMesh(core_axis_name="c", subcore_axis_name="s")
    sc = pl.kernel(
        _sc_body,
        out_type=jax.ShapeDtypeStruct((6, B), jnp.float32),
        mesh=mesh,
        compiler_params=pltpu.CompilerParams(needs_layout_passes=False),
        scratch_types=[
            pltpu.VMEM((_NP * _VOCAB,), jnp.int32),
            pltpu.VMEM((_NP * _VOCAB,), jnp.int32),
            pltpu.VMEM((_PV,), jnp.float32),
            pltpu.VMEM((_CTX, rg), jnp.int32),
            pltpu.VMEM((rg,), jnp.int32),
            pltpu.VMEM((_NEG, rg), jnp.float32),
            pltpu.VMEM((6, rg), jnp.float32),
        ],
    )
    scores = sc(_pack_pairs(ctx_W.T), _pack_pairs(emb_W.T),
                context.T, target, r.T, p_pad)

    loss = pl.pallas_call(
        _tc_body,
        out_shape=jax.ShapeDtypeStruct((1, 1), jnp.float32),
    )(scores)
    return loss[0, 0]
